# baseline (speedup 1.0000x reference)
import functools

import jax
import jax.numpy as jnp
from jax import lax
from jax.experimental import pallas as pl
from jax.experimental.pallas import tpu as pltpu
from jax.experimental.pallas import tpu_sc as plsc

_T = 32768
_H = 768
_E = 8
_BLK = 2048

_NC, _NS, _L = 2, 16, 16
_NW = _NC * _NS
_TPW = _T // _NW
_G = _TPW // _L


def _gate_body(x_ref, w_ref, o_ref):
    o_ref[...] = lax.dot_general(
        w_ref[...], x_ref[...],
        (((1,), (1,)), ((), ())),
        preferred_element_type=jnp.float32,
    )


def _gate_logits_t(x, W):
    return pl.pallas_call(
        _gate_body,
        grid=(_T // _BLK,),
        in_specs=[
            pl.BlockSpec((_BLK, _H), lambda i: (i, 0)),
            pl.BlockSpec((_E, _H), lambda i: (0, 0)),
        ],
        out_specs=pl.BlockSpec((_E, _BLK), lambda i: (0, i)),
        out_shape=jax.ShapeDtypeStruct((_E, _T), jnp.float32),
    )(x, W)


def _route_body(lg_hbm, w_hbm, i_hbm, l_v, ow_v, oi_v):
    wid = lax.axis_index("s") * _NC + lax.axis_index("c")
    base = wid * _TPW
    pltpu.sync_copy(lg_hbm.at[:, pl.ds(base, _TPW)], l_v)

    def body(g, carry):
        lane = lax.iota(jnp.int32, _L)
        col = g * _L
        s = [l_v[e, pl.ds(col, _L)] for e in range(_E)]
        m1 = s[0]
        i1 = jnp.zeros((_L,), jnp.int32)
        for e in range(1, _E):
            gt = s[e] > m1
            m1 = jnp.where(gt, s[e], m1)
            i1 = jnp.where(gt, e, i1)
        neg = jnp.float32(-3.0e38)
        m2 = jnp.full((_L,), neg, jnp.float32)
        i2 = jnp.zeros((_L,), jnp.int32)
        for e in range(_E):
            cand = jnp.where(i1 == e, neg, s[e])
            gt = cand > m2
            m2 = jnp.where(gt, cand, m2)
            i2 = jnp.where(gt, e, i2)
        t = jnp.exp(m2 - m1)
        ssq = 1.0 + t * t
        bi = lax.bitcast_convert_type(ssq, jnp.int32)
        bi = 0x5F3759DF - (bi >> 1)
        y = lax.bitcast_convert_type(bi, jnp.float32)
        for _ in range(3):
            y = y * (1.5 - 0.5 * ssq * y * y)
        w1 = 2.0 * y
        w2 = 2.0 * t * y
        pos = 2 * col + 2 * lane
        plsc.store_scatter(ow_v, [pos], w1)
        plsc.store_scatter(ow_v, [pos + 1], w2)
        plsc.store_scatter(oi_v, [pos], i1)
        plsc.store_scatter(oi_v, [pos + 1], i2)
        return carry

    lax.fori_loop(0, _G, body, 0)
    pltpu.sync_copy(ow_v, w_hbm.at[pl.ds(2 * base, 2 * _TPW)])
    pltpu.sync_copy(oi_v, i_hbm.at[pl.ds(2 * base, 2 * _TPW)])


@functools.cache
def _route():
    return pl.kernel(
        _route_body,
        out_type=[
            jax.ShapeDtypeStruct((2 * _T,), jnp.float32),
            jax.ShapeDtypeStruct((2 * _T,), jnp.int32),
        ],
        mesh=plsc.VectorSubcoreMesh(
            core_axis_name="c", subcore_axis_name="s",
            num_cores=_NC, num_subcores=_NS,
        ),
        scratch_types=[
            pltpu.VMEM((_E, _TPW), jnp.float32),
            pltpu.VMEM((2 * _TPW,), jnp.float32),
            pltpu.VMEM((2 * _TPW,), jnp.int32),
        ],
        compiler_params=pltpu.CompilerParams(needs_layout_passes=False),
    )


def kernel(x, W):
    logits_t = _gate_logits_t(x, W)
    w_flat, i_flat = _route()(logits_t)
    return w_flat.reshape(_T, 2), i_flat.reshape(_T, 2)


# speedup vs baseline: 1.0946x; 1.0946x over previous
import functools

import jax
import jax.numpy as jnp
from jax import lax
from jax.experimental import pallas as pl
from jax.experimental.pallas import tpu as pltpu
from jax.experimental.pallas import tpu_sc as plsc

_T = 32768
_H = 768
_E = 8
_BLK = 2048

_NC, _NS, _L = 2, 16, 16
_NW = _NC * _NS
_TPW = _T // _NW
_G = _TPW // _L


def _gate_body(x_ref, w_ref, o_ref):
    o_ref[...] = lax.dot_general(
        w_ref[...], x_ref[...],
        (((1,), (1,)), ((), ())),
        preferred_element_type=jnp.float32,
    )


def _gate_logits_t(x, W):
    return pl.pallas_call(
        _gate_body,
        grid=(_T // _BLK,),
        in_specs=[
            pl.BlockSpec((_BLK, _H), lambda i: (i, 0)),
            pl.BlockSpec((_E, _H), lambda i: (0, 0)),
        ],
        out_specs=pl.BlockSpec((_E, _BLK), lambda i: (0, i)),
        out_shape=jax.ShapeDtypeStruct((_E, _T), jnp.float32),
    )(x, W)


def _route_body(lg_hbm, w_hbm, i_hbm, l_v, ow_v, oi_v):
    wid = lax.axis_index("s") * _NC + lax.axis_index("c")
    base = wid * _TPW
    pltpu.sync_copy(lg_hbm.at[:, pl.ds(base, _TPW)], l_v)

    def body(g, carry):
        lane = lax.iota(jnp.int32, _L)
        col = g * _L
        s = [l_v[e, pl.ds(col, _L)] for e in range(_E)]
        m1 = s[0]
        i1 = jnp.zeros((_L,), jnp.int32)
        for e in range(1, _E):
            gt = s[e] > m1
            m1 = jnp.where(gt, s[e], m1)
            i1 = jnp.where(gt, e, i1)
        neg = jnp.float32(-3.0e38)
        m2 = jnp.full((_L,), neg, jnp.float32)
        i2 = jnp.zeros((_L,), jnp.int32)
        for e in range(_E):
            cand = jnp.where(i1 == e, neg, s[e])
            gt = cand > m2
            m2 = jnp.where(gt, cand, m2)
            i2 = jnp.where(gt, e, i2)
        t = jnp.exp(m2 - m1)
        ssq = 1.0 + t * t
        bi = lax.bitcast_convert_type(ssq, jnp.int32)
        bi = 0x5F3759DF - (bi >> 1)
        y = lax.bitcast_convert_type(bi, jnp.float32)
        for _ in range(3):
            y = y * (1.5 - 0.5 * ssq * y * y)
        w1 = 2.0 * y
        w2 = 2.0 * t * y
        row = col + lane
        c0 = jnp.zeros((_L,), jnp.int32)
        c1 = jnp.ones((_L,), jnp.int32)
        plsc.store_scatter(ow_v, [row, c0], w1)
        plsc.store_scatter(ow_v, [row, c1], w2)
        plsc.store_scatter(oi_v, [row, c0], i1)
        plsc.store_scatter(oi_v, [row, c1], i2)
        return carry

    lax.fori_loop(0, _G, body, 0)
    pltpu.sync_copy(ow_v, w_hbm.at[pl.ds(base, _TPW), :])
    pltpu.sync_copy(oi_v, i_hbm.at[pl.ds(base, _TPW), :])


@functools.cache
def _route():
    return pl.kernel(
        _route_body,
        out_type=[
            jax.ShapeDtypeStruct((_T, 2), jnp.float32),
            jax.ShapeDtypeStruct((_T, 2), jnp.int32),
        ],
        mesh=plsc.VectorSubcoreMesh(
            core_axis_name="c", subcore_axis_name="s",
            num_cores=_NC, num_subcores=_NS,
        ),
        scratch_types=[
            pltpu.VMEM((_E, _TPW), jnp.float32),
            pltpu.VMEM((_TPW, 2), jnp.float32),
            pltpu.VMEM((_TPW, 2), jnp.int32),
        ],
        compiler_params=pltpu.CompilerParams(
            needs_layout_passes=False, use_tc_tiling_on_sc=False),
    )


def kernel(x, W):
    logits_t = _gate_logits_t(x, W)
    ew, ei = _route()(logits_t)
    return ew, ei


# speedup vs baseline: 1.8776x; 1.7153x over previous
import functools

import jax
import jax.numpy as jnp
from jax import lax
from jax.experimental import pallas as pl
from jax.experimental.pallas import tpu as pltpu
from jax.experimental.pallas import tpu_sc as plsc

_T = 32768
_H = 768
_E = 8
_BLK = 2048
_TB = _BLK // 128
_NB = _T // 128

_NC, _NS, _L = 2, 16, 16
_NW = _NC * _NS
_TPW = _T // _NW
_BPW = _TPW // 128


def _gate_body(x_ref, w_ref, o_ref):
    lg = lax.dot_general(
        w_ref[...], x_ref[...],
        (((1,), (1,)), ((), ())),
        preferred_element_type=jnp.float32,
    )
    for b in range(_TB):
        o_ref[b] = lg[:, 128 * b:128 * (b + 1)]


def _gate_logits(x, W):
    return pl.pallas_call(
        _gate_body,
        grid=(_T // _BLK,),
        in_specs=[
            pl.BlockSpec((_BLK, _H), lambda i: (i, 0)),
            pl.BlockSpec((_E, _H), lambda i: (0, 0)),
        ],
        out_specs=pl.BlockSpec((_TB, _E, 128), lambda i: (i, 0, 0)),
        out_shape=jax.ShapeDtypeStruct((_NB, _E, 128), jnp.float32),
    )(x, W)


def _route_body(lg_hbm, w1_hbm, w2_hbm, i1_hbm, i2_hbm,
                l_v, ow1_v, ow2_v, oi1_v, oi2_v):
    wid = lax.axis_index("s") * _NC + lax.axis_index("c")
    base = wid * _TPW
    pltpu.sync_copy(lg_hbm.at[pl.ds(wid * _BPW, _BPW)], l_v)

    def body(sub, carry):
        off = sub * _L
        for b in range(_BPW):
            s = [l_v[b, e, pl.ds(off, _L)] for e in range(_E)]
            m1 = s[0]
            i1 = jnp.zeros((_L,), jnp.int32)
            for e in range(1, _E):
                gt = s[e] > m1
                m1 = jnp.where(gt, s[e], m1)
                i1 = jnp.where(gt, e, i1)
            neg = jnp.float32(-3.0e38)
            m2 = jnp.full((_L,), neg, jnp.float32)
            i2 = jnp.zeros((_L,), jnp.int32)
            for e in range(_E):
                cand = jnp.where(i1 == e, neg, s[e])
                gt = cand > m2
                m2 = jnp.where(gt, cand, m2)
                i2 = jnp.where(gt, e, i2)
            t = jnp.exp(m2 - m1)
            ssq = 1.0 + t * t
            bi = lax.bitcast_convert_type(ssq, jnp.int32)
            bi = 0x5F3759DF - (bi >> 1)
            y = lax.bitcast_convert_type(bi, jnp.float32)
            for _ in range(3):
                y = y * (1.5 - 0.5 * ssq * y * y)
            col = b * 128 + off
            ow1_v[pl.ds(col, _L)] = 2.0 * y
            ow2_v[pl.ds(col, _L)] = 2.0 * t * y
            oi1_v[pl.ds(col, _L)] = i1
            oi2_v[pl.ds(col, _L)] = i2
        return carry

    lax.fori_loop(0, 128 // _L, body, 0)
    pltpu.sync_copy(ow1_v, w1_hbm.at[pl.ds(base, _TPW)])
    pltpu.sync_copy(ow2_v, w2_hbm.at[pl.ds(base, _TPW)])
    pltpu.sync_copy(oi1_v, i1_hbm.at[pl.ds(base, _TPW)])
    pltpu.sync_copy(oi2_v, i2_hbm.at[pl.ds(base, _TPW)])


@functools.cache
def _route():
    return pl.kernel(
        _route_body,
        out_type=[
            jax.ShapeDtypeStruct((_T,), jnp.float32),
            jax.ShapeDtypeStruct((_T,), jnp.float32),
            jax.ShapeDtypeStruct((_T,), jnp.int32),
            jax.ShapeDtypeStruct((_T,), jnp.int32),
        ],
        mesh=plsc.VectorSubcoreMesh(
            core_axis_name="c", subcore_axis_name="s",
            num_cores=_NC, num_subcores=_NS,
        ),
        scratch_types=[
            pltpu.VMEM((_BPW, _E, 128), jnp.float32),
            pltpu.VMEM((_TPW,), jnp.float32),
            pltpu.VMEM((_TPW,), jnp.float32),
            pltpu.VMEM((_TPW,), jnp.int32),
            pltpu.VMEM((_TPW,), jnp.int32),
        ],
        compiler_params=pltpu.CompilerParams(
            needs_layout_passes=False, use_tc_tiling_on_sc=False),
    )


def kernel(x, W):
    logits = _gate_logits(x, W)
    w1, w2, i1, i2 = _route()(logits)
    ew = jnp.stack([w1, w2], axis=-1)
    ei = jnp.stack([i1, i2], axis=-1)
    return ew, ei
